# Initial kernel scaffold; baseline (speedup 1.0000x reference)
import jax
import jax.numpy as jnp
from jax.experimental import pallas as pl


def kernel(node_feat, edge_index, emb, W1, b1, W2, b2):
    raise NotImplementedError("write your pallas kernel here")


# speedup vs baseline: 8.7844x; 8.7844x over previous
import functools

import jax
import jax.numpy as jnp
from jax import lax
from jax.experimental import pallas as pl
from jax.experimental.pallas import tpu as pltpu
from jax.experimental.pallas import tpu_sc as plsc

N = 10000
E = 320000
D = 128
V = 128

NP = 10240
CHUNK = 128
NCHUNKS = 2528
EP = NCHUNKS * CHUNK
NC, NS = 2, 16
NW = NC * NS
CPW = NCHUNKS // NW
CPT = NCHUNKS // NS
RPT = NP // NS
CELL = NP * V // NS
ZB = 8192

_mesh = plsc.VectorSubcoreMesh(core_axis_name="c", subcore_axis_name="s")
_f32 = jnp.float32


@functools.partial(
    pl.kernel,
    out_type=(jax.ShapeDtypeStruct((NP,), _f32),
              jax.ShapeDtypeStruct((NP,), _f32)),
    mesh=_mesh,
    compiler_params=pltpu.CompilerParams(use_tc_tiling_on_sc=False, needs_layout_passes=False),
    scratch_types=[
        pltpu.VMEM((CPT, CHUNK), jnp.int32),
        pltpu.VMEM((CHUNK,), _f32),
        pltpu.VMEM((RPT,), _f32),
        pltpu.VMEM_SHARED((NP,), _f32),
    ],
)
def _degree_kernel(src_hbm, dst_hbm, dout_hbm, din_hbm, idxblk, ones, zb, hist):
    cid = lax.axis_index("c")
    sid = lax.axis_index("s")

    def fill_z(i, _):
        zb[pl.ds(i * 16, 16)] = jnp.zeros((16,), _f32)
        return 0

    lax.fori_loop(0, RPT // 16, fill_z, 0)

    def fill_o(i, _):
        ones[pl.ds(i * 16, 16)] = jnp.ones((16,), _f32)
        return 0

    lax.fori_loop(0, CHUNK // 16, fill_o, 0)
    pltpu.sync_copy(zb, hist.at[pl.ds(sid * RPT, RPT)])

    @pl.when(cid == 0)
    def _():
        pltpu.sync_copy(src_hbm.at[pl.ds(sid * CPT, CPT)], idxblk)

    @pl.when(cid == 1)
    def _():
        pltpu.sync_copy(dst_hbm.at[pl.ds(sid * CPT, CPT)], idxblk)

    plsc.subcore_barrier()

    def body(j, _):
        pltpu.sync_copy(ones, hist.at[idxblk.at[j]], add=True)
        return 0

    lax.fori_loop(0, CPT, body, 0)
    plsc.subcore_barrier()

    @pl.when(cid == 0)
    def _():
        pltpu.sync_copy(hist.at[pl.ds(sid * RPT, RPT)],
                        dout_hbm.at[pl.ds(sid * RPT, RPT)])

    @pl.when(cid == 1)
    def _():
        pltpu.sync_copy(hist.at[pl.ds(sid * RPT, RPT)],
                        din_hbm.at[pl.ds(sid * RPT, RPT)])


@functools.partial(
    pl.kernel,
    out_type=jax.ShapeDtypeStruct((NC, NP * V), _f32),
    mesh=_mesh,
    compiler_params=pltpu.CompilerParams(use_tc_tiling_on_sc=False, needs_layout_passes=False),
    scratch_types=[
        pltpu.VMEM((CPW, CHUNK), jnp.int32),
        pltpu.VMEM((CPW, CHUNK), jnp.int32),
        pltpu.VMEM((NP,), jnp.int32),
        pltpu.VMEM((NP,), _f32),
        pltpu.VMEM((CHUNK,), jnp.int32),
        pltpu.VMEM((CHUNK,), _f32),
        pltpu.VMEM((ZB,), _f32),
        pltpu.VMEM_SHARED((NP * V,), _f32),
    ],
)
def _hist_kernel(src_hbm, dst_hbm, nf_hbm, no_hbm, cout_hbm,
                 sidx, didx, nf_ts, no_ts, fidx, fval, zb, csh):
    cid = lax.axis_index("c")
    sid = lax.axis_index("s")

    def fill_z(i, _):
        zb[pl.ds(i * 16, 16)] = jnp.zeros((16,), _f32)
        return 0

    lax.fori_loop(0, ZB // 16, fill_z, 0)

    def zcopy(i, _):
        pltpu.sync_copy(zb, csh.at[pl.ds(sid * CELL + i * ZB, ZB)])
        return 0

    lax.fori_loop(0, CELL // ZB, zcopy, 0)
    pltpu.sync_copy(nf_hbm, nf_ts)
    pltpu.sync_copy(no_hbm, no_ts)
    base = (cid * NS + sid) * CPW
    pltpu.sync_copy(src_hbm.at[pl.ds(base, CPW)], sidx)
    pltpu.sync_copy(dst_hbm.at[pl.ds(base, CPW)], didx)
    plsc.subcore_barrier()

    def chunk_body(j, _):
        for i in range(CHUNK // 16):
            s16 = sidx[j, pl.ds(i * 16, 16)]
            d16 = didx[j, pl.ds(i * 16, 16)]
            v16 = plsc.load_gather(nf_ts, [s16])
            w16 = plsc.load_gather(no_ts, [s16])
            fidx[pl.ds(i * 16, 16)] = d16 * V + v16
            fval[pl.ds(i * 16, 16)] = w16
        pltpu.sync_copy(fval, csh.at[fidx], add=True)
        return 0

    lax.fori_loop(0, CPW, chunk_body, 0)
    plsc.subcore_barrier()
    pltpu.sync_copy(csh.at[pl.ds(sid * CELL, CELL)],
                    cout_hbm.at[cid, pl.ds(sid * CELL, CELL)])


@functools.partial(
    pl.kernel,
    out_type=jax.ShapeDtypeStruct((NC, NP, D), _f32),
    mesh=_mesh,
    compiler_params=pltpu.CompilerParams(use_tc_tiling_on_sc=False, needs_layout_passes=False),
    scratch_types=[
        pltpu.VMEM((CPW, CHUNK), jnp.int32),
        pltpu.VMEM((CPW, CHUNK), jnp.int32),
        pltpu.VMEM((CHUNK, D), _f32),
        pltpu.VMEM((64, D), _f32),
        pltpu.VMEM_SHARED((NP, D), _f32),
        pltpu.SemaphoreType.DMA,
    ],
)
def _spmm_kernel(src_hbm, dst_hbm, x_hbm, agg_out,
                 sidx, didx, rows, zb2, aggsh, sem):
    cid = lax.axis_index("c")
    sid = lax.axis_index("s")

    def fill_z(r, _):
        for i in range(D // 16):
            zb2[r, pl.ds(i * 16, 16)] = jnp.zeros((16,), _f32)
        return 0

    lax.fori_loop(0, 64, fill_z, 0)

    def zcopy(i, _):
        pltpu.sync_copy(zb2, aggsh.at[pl.ds(sid * RPT + i * 64, 64)])
        return 0

    lax.fori_loop(0, RPT // 64, zcopy, 0)
    base = (cid * NS + sid) * CPW
    pltpu.sync_copy(src_hbm.at[pl.ds(base, CPW)], sidx)
    pltpu.sync_copy(dst_hbm.at[pl.ds(base, CPW)], didx)
    plsc.subcore_barrier()

    def body(j, _):
        pltpu.async_copy(x_hbm.at[sidx.at[j]], rows, sem).wait()
        pltpu.sync_copy(rows, aggsh.at[didx.at[j]], add=True)
        return 0

    lax.fori_loop(0, CPW, body, 0)
    plsc.subcore_barrier()
    pltpu.sync_copy(aggsh.at[pl.ds(sid * RPT, RPT)],
                    agg_out.at[cid, pl.ds(sid * RPT, RPT)])


def _prep_body(dout_ref, din_ref, emb_ref, w1_ref, no_ref, ni_ref, ew1_ref):
    do = dout_ref[...]
    di = din_ref[...]
    no_ref[...] = jnp.where(do > 0, lax.rsqrt(do), 0.0)
    ni_ref[...] = jnp.where(di > 0, lax.rsqrt(di), 0.0)
    ew1_ref[...] = jnp.dot(emb_ref[...], w1_ref[...],
                           preferred_element_type=_f32)


_prep_kernel = pl.pallas_call(
    _prep_body,
    out_shape=(jax.ShapeDtypeStruct((NP // 128, 128), _f32),
               jax.ShapeDtypeStruct((NP // 128, 128), _f32),
               jax.ShapeDtypeStruct((V, D), _f32)),
)

_RB = 1024


def _h1_body(c_ref, ew1_ref, b1_ref, no_ref, ni_ref, h1s_ref):
    cs = c_ref[0] + c_ref[1]
    z = ni_ref[...] * jnp.dot(cs, ew1_ref[...],
                              preferred_element_type=_f32) + b1_ref[...]
    h1s_ref[...] = no_ref[...] * jnp.maximum(z, 0.0)


_h1_kernel = pl.pallas_call(
    _h1_body,
    grid=(NP // _RB,),
    in_specs=[
        pl.BlockSpec((NC, _RB, V), lambda i: (0, i, 0)),
        pl.BlockSpec((V, D), lambda i: (0, 0)),
        pl.BlockSpec((1, D), lambda i: (0, 0)),
        pl.BlockSpec((_RB, 1), lambda i: (i, 0)),
        pl.BlockSpec((_RB, 1), lambda i: (i, 0)),
    ],
    out_specs=pl.BlockSpec((_RB, D), lambda i: (i, 0)),
    out_shape=jax.ShapeDtypeStruct((NP, D), _f32),
)


def _final_body(a_ref, w2_ref, b2_ref, ni_ref, out_ref):
    i = pl.program_id(0)

    @pl.when(i == 0)
    def _():
        out_ref[...] = jnp.zeros((8, D), _f32)

    a = a_ref[0] + a_ref[1]
    h2 = jnp.maximum(
        jnp.dot(ni_ref[...] * a, w2_ref[...],
                preferred_element_type=_f32) + b2_ref[...], 0.0)
    rowid = i * _RB + lax.broadcasted_iota(jnp.int32, (_RB, 1), 0)
    h2 = jnp.where(rowid < N, h2, 0.0)
    out_ref[...] += jnp.sum(h2.reshape(_RB // 8, 8, D), axis=0)

    @pl.when(i == NP // _RB - 1)
    def _():
        out_ref[...] = jnp.broadcast_to(
            jnp.sum(out_ref[...], axis=0, keepdims=True) * (1.0 / N), (8, D))


_final_kernel = pl.pallas_call(
    _final_body,
    grid=(NP // _RB,),
    in_specs=[
        pl.BlockSpec((NC, _RB, D), lambda i: (0, i, 0)),
        pl.BlockSpec((D, D), lambda i: (0, 0)),
        pl.BlockSpec((1, D), lambda i: (0, 0)),
        pl.BlockSpec((_RB, 1), lambda i: (i, 0)),
    ],
    out_specs=pl.BlockSpec((8, D), lambda i: (0, 0)),
    out_shape=jax.ShapeDtypeStruct((8, D), _f32),
)


def kernel(node_feat, edge_index, emb, W1, b1, W2, b2):
    pad = jnp.full((EP - E,), NP - 1, jnp.int32)
    src = jnp.concatenate([edge_index[0].astype(jnp.int32), pad])
    dst = jnp.concatenate([edge_index[1].astype(jnp.int32), pad])
    src = src.reshape(NCHUNKS, CHUNK)
    dst = dst.reshape(NCHUNKS, CHUNK)
    nfp = jnp.concatenate(
        [node_feat.astype(jnp.int32), jnp.zeros((NP - N,), jnp.int32)])

    dout, din = _degree_kernel(src, dst)
    no2d, ni2d, ew1 = _prep_kernel(dout.reshape(NP // 128, 128),
                                   din.reshape(NP // 128, 128), emb, W1)
    no_flat = no2d.reshape(NP)
    no_col = no2d.reshape(NP, 1)
    ni_col = ni2d.reshape(NP, 1)

    cpart = _hist_kernel(src, dst, nfp, no_flat)
    h1s = _h1_kernel(cpart.reshape(NC, NP, V), ew1, b1.reshape(1, D),
                     no_col, ni_col)
    agg = _spmm_kernel(src, dst, h1s)
    hsum = _final_kernel(agg, W2, b2.reshape(1, D), ni_col)
    return hsum[0:1, :]
